# Initial kernel scaffold; baseline (speedup 1.0000x reference)
import jax
import jax.numpy as jnp
from jax.experimental import pallas as pl


def kernel(nodes, positions, query_positions, params):
    raise NotImplementedError("write your pallas kernel here")


# speedup vs baseline: 4.5698x; 4.5698x over previous
import functools

import jax
import jax.numpy as jnp
from jax import lax
from jax.experimental import pallas as pl
from jax.experimental.pallas import tpu as pltpu
from jax.experimental.pallas import tpu_sc as plsc

NQ = 10000
NS = 10000
K = 3
ND = 128
ED = 16
HID = 128
PPAD = 128

_pallas_call = pl.pallas_call

QB_KNN = 200


def _knn_body(qp_ref, qb_ref, spt_ref, sbt_ref, idx_ref, dsel_ref):
    qp = qp_ref[...]
    qb = qb_ref[...]
    spt = spt_ref[...]
    sbt = sbt_ref[...]
    sumq = jnp.sum(qp * qp, axis=1, keepdims=True)
    sums = jnp.sum(spt * spt, axis=0, keepdims=True)
    dot = lax.dot_general(qb, sbt, (((1,), (0,)), ((), ())),
                          preferred_element_type=jnp.float32)
    d2 = (sumq + sums) - 2.0 * dot
    dist = jnp.sqrt(jnp.maximum(d2, 1e-12))
    iota = lax.broadcasted_iota(jnp.int32, dist.shape, 1)
    dw = dist
    for k in range(K):
        m = jnp.min(dw, axis=1, keepdims=True)
        i = jnp.min(jnp.where(dw == m, iota, jnp.int32(2 ** 30)),
                    axis=1, keepdims=True)
        idx_ref[:, k:k + 1] = i
        dsel_ref[:, k:k + 1] = m
        if k < K - 1:
            dw = jnp.where(iota == i, jnp.float32(jnp.inf), dw)


def _knn(qpos, spos):
    qb = qpos.astype(jnp.bfloat16)
    sb = spos.astype(jnp.bfloat16)
    qb = jnp.pad(qb, ((0, 0), (0, PPAD - qb.shape[1])))
    sbt = jnp.pad(sb.T, ((0, PPAD - sb.shape[1]), (0, 0)))
    grid = (NQ // QB_KNN,)
    idx, dsel = _pallas_call(
        _knn_body,
        grid=grid,
        in_specs=[
            pl.BlockSpec((QB_KNN, 3), lambda i: (i, 0)),
            pl.BlockSpec((QB_KNN, PPAD), lambda i: (i, 0)),
            pl.BlockSpec((3, NS), lambda i: (0, 0)),
            pl.BlockSpec((PPAD, NS), lambda i: (0, 0)),
        ],
        out_specs=[
            pl.BlockSpec((QB_KNN, K), lambda i: (i, 0)),
            pl.BlockSpec((QB_KNN, K), lambda i: (i, 0)),
        ],
        out_shape=[
            jax.ShapeDtypeStruct((NQ, K), jnp.int32),
            jax.ShapeDtypeStruct((NQ, K), jnp.float32),
        ],
    )(qpos, qb, spos.T, sbt)
    return idx, dsel


BPAD = 32768
CHUNK = 128


def _gather_rows(table, idx2d):
    info = plsc.get_sparse_core_info()
    nc, nsub = info.num_cores, info.num_subcores
    nw = nc * nsub
    cpw = BPAD // nw // CHUNK
    mesh = plsc.VectorSubcoreMesh(core_axis_name="c", subcore_axis_name="s")

    @functools.partial(
        pl.kernel, mesh=mesh,
        out_type=jax.ShapeDtypeStruct((BPAD, ND), jnp.float32),
        scratch_types=[
            pltpu.VMEM((cpw, CHUNK), jnp.int32),
            pltpu.VMEM((CHUNK, ND), jnp.float32),
            pltpu.SemaphoreType.DMA,
        ],
    )
    def gk(table_hbm, idx_hbm, out_hbm, idx_v, rows_v, sem):
        wid = lax.axis_index("s") * nc + lax.axis_index("c")
        rowbase = wid * cpw
        pltpu.sync_copy(idx_hbm.at[pl.ds(rowbase, cpw)], idx_v)
        for j in range(cpw):
            pltpu.async_copy(table_hbm.at[idx_v.at[j]], rows_v, sem).wait()
            pltpu.sync_copy(rows_v,
                            out_hbm.at[pl.ds((rowbase + j) * CHUNK, CHUNK)])

    return gk(table, idx2d)


QB = 1000


_SQRT_HALF = 0.7071067811865476


def _gelu(x):
    return 0.5 * x * (1.0 + lax.erf(x * _SQRT_HALF))


def _init_enc_body(g_ref, d_ref, wbar_ref, b1_ref, w2_ref, b2_ref,
                   pn_ref, e_ref):
    g = g_ref[...]
    pn_ref[...] = ((g[0] + g[1]) + g[2]) / 3.0
    wbar = wbar_ref[...]
    b1 = b1_ref[...]
    w2 = w2_ref[...]
    b2 = b2_ref[...]
    for k in range(K):
        dk = d_ref[k]
        h = _gelu(dk * wbar + b1)
        e_ref[k] = jnp.dot(h, w2, preferred_element_type=jnp.float32) + b2


def _init_enc(g, dsel_t, wbar, b1, w2, b2):
    grid = (NQ // QB,)
    pn, e = _pallas_call(
        _init_enc_body,
        grid=grid,
        in_specs=[
            pl.BlockSpec((K, QB, ND), lambda i: (0, i, 0)),
            pl.BlockSpec((K, QB, 1), lambda i: (0, i, 0)),
            pl.BlockSpec((1, HID), lambda i: (0, 0)),
            pl.BlockSpec((1, HID), lambda i: (0, 0)),
            pl.BlockSpec((HID, ED), lambda i: (0, 0)),
            pl.BlockSpec((1, ED), lambda i: (0, 0)),
        ],
        out_specs=[
            pl.BlockSpec((QB, ND), lambda i: (i, 0)),
            pl.BlockSpec((K, QB, ED), lambda i: (0, i, 0)),
        ],
        out_shape=[
            jax.ShapeDtypeStruct((NQ, ND), jnp.float32),
            jax.ShapeDtypeStruct((K, NQ, ED), jnp.float32),
        ],
    )(g, dsel_t, wbar, b1, w2, b2)
    return pn, e


def _layer_body(pn_ref, sf_ref, e_ref, w1s_ref, w1r_ref, w1e_ref, b1_ref,
                w2_ref, b2_ref, wn1p_ref, wn1a_ref, bn1_ref, wn2_ref, bn2_ref,
                pno_ref, eo_ref):
    pn = pn_ref[...]
    rterm = (jnp.dot(pn, w1r_ref[...], preferred_element_type=jnp.float32)
             + b1_ref[...])
    en = []
    for k in range(K):
        h = _gelu(jnp.dot(sf_ref[k], w1s_ref[...],
                          preferred_element_type=jnp.float32)
                  + jnp.dot(e_ref[k], w1e_ref[...],
                            preferred_element_type=jnp.float32)
                  + rterm)
        enk = (jnp.dot(h, w2_ref[...], preferred_element_type=jnp.float32)
               + b2_ref[...])
        en.append(enk)
        eo_ref[k] = e_ref[k] + enk
    agg = (en[0] + en[1]) + en[2]
    hn = _gelu(jnp.dot(pn, wn1p_ref[...], preferred_element_type=jnp.float32)
               + jnp.dot(agg, wn1a_ref[...], preferred_element_type=jnp.float32)
               + bn1_ref[...])
    pno_ref[...] = pn + (jnp.dot(hn, wn2_ref[...],
                                 preferred_element_type=jnp.float32)
                         + bn2_ref[...])


def _layer(pn, sf, e, lw):
    grid = (NQ // QB,)
    wspecs = [pl.BlockSpec(w.shape, lambda i: tuple(0 for _ in w.shape))
              for w in lw]
    pno, eo = _pallas_call(
        _layer_body,
        grid=grid,
        in_specs=[
            pl.BlockSpec((QB, ND), lambda i: (i, 0)),
            pl.BlockSpec((K, QB, ND), lambda i: (0, i, 0)),
            pl.BlockSpec((K, QB, ED), lambda i: (0, i, 0)),
        ] + wspecs,
        out_specs=[
            pl.BlockSpec((QB, ND), lambda i: (i, 0)),
            pl.BlockSpec((K, QB, ED), lambda i: (0, i, 0)),
        ],
        out_shape=[
            jax.ShapeDtypeStruct((NQ, ND), jnp.float32),
            jax.ShapeDtypeStruct((K, NQ, ED), jnp.float32),
        ],
    )(pn, sf, e, *lw)
    return pno, eo


def _layer2_out_body(pn_ref, sf_ref, e_ref, w1s_ref, w1r_ref, w1e_ref,
                     b1_ref, w2_ref, b2_ref, wn1p_ref, wn1a_ref, bn1_ref,
                     wn2_ref, bn2_ref, wo1_ref, bo1_ref, wo2_ref, bo2_ref,
                     wo3_ref, bo3_ref, out_ref):
    pn = pn_ref[...]
    rterm = (jnp.dot(pn, w1r_ref[...], preferred_element_type=jnp.float32)
             + b1_ref[...])
    en = []
    for k in range(K):
        h = _gelu(jnp.dot(sf_ref[k], w1s_ref[...],
                          preferred_element_type=jnp.float32)
                  + jnp.dot(e_ref[k], w1e_ref[...],
                            preferred_element_type=jnp.float32)
                  + rterm)
        en.append(jnp.dot(h, w2_ref[...], preferred_element_type=jnp.float32)
                  + b2_ref[...])
    agg = (en[0] + en[1]) + en[2]
    hn = _gelu(jnp.dot(pn, wn1p_ref[...], preferred_element_type=jnp.float32)
               + jnp.dot(agg, wn1a_ref[...], preferred_element_type=jnp.float32)
               + bn1_ref[...])
    po = pn + (jnp.dot(hn, wn2_ref[...], preferred_element_type=jnp.float32)
               + bn2_ref[...])
    h1 = _gelu(jnp.dot(po, wo1_ref[...], preferred_element_type=jnp.float32)
               + bo1_ref[...])
    h2 = _gelu(jnp.dot(h1, wo2_ref[...], preferred_element_type=jnp.float32)
               + bo2_ref[...])
    out_ref[...] = (jnp.dot(h2, wo3_ref[...],
                            preferred_element_type=jnp.float32) + bo3_ref[...])


def _layer2_out(pn, sf, e, lw):
    grid = (NQ // QB,)
    wspecs = [pl.BlockSpec(w.shape, lambda i: tuple(0 for _ in w.shape))
              for w in lw]
    out = _pallas_call(
        _layer2_out_body,
        grid=grid,
        in_specs=[
            pl.BlockSpec((QB, ND), lambda i: (i, 0)),
            pl.BlockSpec((K, QB, ND), lambda i: (0, i, 0)),
            pl.BlockSpec((K, QB, ED), lambda i: (0, i, 0)),
        ] + wspecs,
        out_specs=pl.BlockSpec((QB, 3), lambda i: (i, 0)),
        out_shape=jax.ShapeDtypeStruct((NQ, 3), jnp.float32),
    )(pn, sf, e, *lw)
    return out


def kernel(nodes, positions, query_positions, params):
    idx, dsel = _knn(query_positions, positions)

    senders = idx.T.reshape(-1)
    sz = jnp.zeros((BPAD - K * NQ,), jnp.int32)
    idx2d = jnp.concatenate([senders, sz]).reshape(BPAD // CHUNK, CHUNK)
    dsel_t = dsel.T.reshape(K, NQ, 1)

    r2 = lambda b: b.reshape(1, -1)

    g = _gather_rows(nodes, idx2d)[:K * NQ].reshape(K, NQ, ND)
    wbar = jnp.sum(params["enc_W"][0], axis=0, keepdims=True)
    pn, e = _init_enc(g, dsel_t, wbar, r2(params["enc_b"][0]),
                      params["enc_W"][1], r2(params["enc_b"][1]))

    lws = []
    for lp in params["layers"]:
        w1 = lp["eW"][0]
        wn1 = lp["nW"][0]
        lws.append([w1[:ND], w1[ND:2 * ND], w1[2 * ND:], r2(lp["eb"][0]),
                    lp["eW"][1], r2(lp["eb"][1]),
                    wn1[:ND], wn1[ND:], r2(lp["nb"][0]),
                    lp["nW"][1], r2(lp["nb"][1])])

    sf = _gather_rows(pn, idx2d)[:K * NQ].reshape(K, NQ, ND)
    pn, e = _layer(pn, sf, e, lws[0])

    sf = _gather_rows(pn, idx2d)[:K * NQ].reshape(K, NQ, ND)
    ow = [params["out_W"][0], r2(params["out_b"][0]),
          params["out_W"][1], r2(params["out_b"][1]),
          params["out_W"][2], r2(params["out_b"][2])]
    return _layer2_out(pn, sf, e, lws[1] + ow)


# speedup vs baseline: 6.0731x; 1.3290x over previous
import functools

import jax
import jax.numpy as jnp
from jax import lax
from jax.experimental import pallas as pl
from jax.experimental.pallas import tpu as pltpu
from jax.experimental.pallas import tpu_sc as plsc

NQ = 10000
NS = 10000
K = 3
ND = 128
ED = 16
HID = 128
PPAD = 128

_pallas_call = pl.pallas_call

QB_KNN = 200


def _knn_body(qp_ref, spt_ref, idx_ref, dsel_ref):
    qp = qp_ref[...]
    spt = spt_ref[...]
    qb = qp.astype(jnp.bfloat16)
    sbt = spt.astype(jnp.bfloat16)
    sumq = jnp.sum(qp * qp, axis=1, keepdims=True)
    sums = jnp.sum(spt * spt, axis=0, keepdims=True)
    dot = lax.dot_general(qb, sbt, (((1,), (0,)), ((), ())),
                          preferred_element_type=jnp.float32)
    d2 = (sumq + sums) - 2.0 * dot
    dist = jnp.sqrt(jnp.maximum(d2, 1e-12))
    iota = lax.broadcasted_iota(jnp.int32, dist.shape, 1)
    dw = dist
    for k in range(K):
        m = jnp.min(dw, axis=1, keepdims=True)
        i = jnp.min(jnp.where(dw == m, iota, jnp.int32(2 ** 30)),
                    axis=1, keepdims=True)
        idx_ref[:, k:k + 1] = i
        dsel_ref[:, k:k + 1] = m
        if k < K - 1:
            dw = jnp.where(iota == i, jnp.float32(jnp.inf), dw)


def _knn(qpos, spos):
    grid = (NQ // QB_KNN,)
    idx, dsel = _pallas_call(
        _knn_body,
        grid=grid,
        in_specs=[
            pl.BlockSpec((QB_KNN, 3), lambda i: (i, 0)),
            pl.BlockSpec((3, NS), lambda i: (0, 0)),
        ],
        out_specs=[
            pl.BlockSpec((QB_KNN, K), lambda i: (i, 0)),
            pl.BlockSpec((QB_KNN, K), lambda i: (i, 0)),
        ],
        out_shape=[
            jax.ShapeDtypeStruct((NQ, K), jnp.int32),
            jax.ShapeDtypeStruct((NQ, K), jnp.float32),
        ],
    )(qpos, spos.T)
    return idx, dsel


BPAD = 30720
CHUNK = 120


def _gather_rows(table, idx2d):
    info = plsc.get_sparse_core_info()
    nc, nsub = info.num_cores, info.num_subcores
    nw = nc * nsub
    cpw = BPAD // nw // CHUNK
    mesh = plsc.VectorSubcoreMesh(core_axis_name="c", subcore_axis_name="s")

    @functools.partial(
        pl.kernel, mesh=mesh,
        out_type=jax.ShapeDtypeStruct((BPAD, ND), jnp.float32),
        scratch_types=[
            pltpu.VMEM((cpw, CHUNK), jnp.int32),
            pltpu.VMEM((cpw, CHUNK, ND), jnp.float32),
            pltpu.SemaphoreType.DMA,
            pltpu.SemaphoreType.DMA,
        ],
    )
    def gk(table_hbm, idx_hbm, out_hbm, idx_v, rows_v, sem_g, sem_o):
        wid = lax.axis_index("s") * nc + lax.axis_index("c")
        rowbase = wid * cpw
        pltpu.sync_copy(idx_hbm.at[pl.ds(rowbase, cpw)], idx_v)
        gets = [pltpu.async_copy(table_hbm.at[idx_v.at[j]], rows_v.at[j],
                                 sem_g) for j in range(cpw)]
        for c in gets:
            c.wait()
        puts = [pltpu.async_copy(rows_v.at[j],
                                 out_hbm.at[pl.ds((rowbase + j) * CHUNK,
                                                  CHUNK)],
                                 sem_o) for j in range(cpw)]
        for c in puts:
            c.wait()

    return gk(table, idx2d)


QB = 1000


_SQRT_HALF = 0.7071067811865476


def _gelu(x):
    return 0.5 * x * (1.0 + lax.erf(x * _SQRT_HALF))


def _init_enc_body(g_ref, d_ref, wbar_ref, b1_ref, w2_ref, b2_ref,
                   pn_ref, e_ref):
    g = g_ref[...]
    pn_ref[...] = ((g[0] + g[1]) + g[2]) / 3.0
    wbar = wbar_ref[...]
    b1 = b1_ref[...]
    w2 = w2_ref[...]
    b2 = b2_ref[...]
    for k in range(K):
        dk = d_ref[k]
        h = _gelu(dk * wbar + b1)
        e_ref[k] = jnp.dot(h, w2, preferred_element_type=jnp.float32) + b2


def _init_enc(g, dsel_t, wbar, b1, w2, b2):
    grid = (NQ // QB,)
    pn, e = _pallas_call(
        _init_enc_body,
        grid=grid,
        in_specs=[
            pl.BlockSpec((K, QB, ND), lambda i: (0, i, 0)),
            pl.BlockSpec((K, QB, 1), lambda i: (0, i, 0)),
            pl.BlockSpec((1, HID), lambda i: (0, 0)),
            pl.BlockSpec((1, HID), lambda i: (0, 0)),
            pl.BlockSpec((HID, ED), lambda i: (0, 0)),
            pl.BlockSpec((1, ED), lambda i: (0, 0)),
        ],
        out_specs=[
            pl.BlockSpec((QB, ND), lambda i: (i, 0)),
            pl.BlockSpec((K, QB, ED), lambda i: (0, i, 0)),
        ],
        out_shape=[
            jax.ShapeDtypeStruct((NQ, ND), jnp.float32),
            jax.ShapeDtypeStruct((K, NQ, ED), jnp.float32),
        ],
    )(g, dsel_t, wbar, b1, w2, b2)
    return pn, e


def _layer_body(pn_ref, sf_ref, e_ref, w1s_ref, w1r_ref, w1e_ref, b1_ref,
                w2_ref, b2_ref, wn1p_ref, wn1a_ref, bn1_ref, wn2_ref, bn2_ref,
                pno_ref, eo_ref):
    pn = pn_ref[...]
    rterm = (jnp.dot(pn, w1r_ref[...], preferred_element_type=jnp.float32)
             + b1_ref[...])
    en = []
    for k in range(K):
        h = _gelu(jnp.dot(sf_ref[k], w1s_ref[...],
                          preferred_element_type=jnp.float32)
                  + jnp.dot(e_ref[k], w1e_ref[...],
                            preferred_element_type=jnp.float32)
                  + rterm)
        enk = (jnp.dot(h, w2_ref[...], preferred_element_type=jnp.float32)
               + b2_ref[...])
        en.append(enk)
        eo_ref[k] = e_ref[k] + enk
    agg = (en[0] + en[1]) + en[2]
    hn = _gelu(jnp.dot(pn, wn1p_ref[...], preferred_element_type=jnp.float32)
               + jnp.dot(agg, wn1a_ref[...], preferred_element_type=jnp.float32)
               + bn1_ref[...])
    pno_ref[...] = pn + (jnp.dot(hn, wn2_ref[...],
                                 preferred_element_type=jnp.float32)
                         + bn2_ref[...])


def _layer(pn, sf, e, lw):
    grid = (NQ // QB,)
    wspecs = [pl.BlockSpec(w.shape, lambda i: tuple(0 for _ in w.shape))
              for w in lw]
    pno, eo = _pallas_call(
        _layer_body,
        grid=grid,
        in_specs=[
            pl.BlockSpec((QB, ND), lambda i: (i, 0)),
            pl.BlockSpec((K, QB, ND), lambda i: (0, i, 0)),
            pl.BlockSpec((K, QB, ED), lambda i: (0, i, 0)),
        ] + wspecs,
        out_specs=[
            pl.BlockSpec((QB, ND), lambda i: (i, 0)),
            pl.BlockSpec((K, QB, ED), lambda i: (0, i, 0)),
        ],
        out_shape=[
            jax.ShapeDtypeStruct((NQ, ND), jnp.float32),
            jax.ShapeDtypeStruct((K, NQ, ED), jnp.float32),
        ],
    )(pn, sf, e, *lw)
    return pno, eo


def _layer2_out_body(pn_ref, sf_ref, e_ref, w1s_ref, w1r_ref, w1e_ref,
                     b1_ref, w2_ref, b2_ref, wn1p_ref, wn1a_ref, bn1_ref,
                     wn2_ref, bn2_ref, wo1_ref, bo1_ref, wo2_ref, bo2_ref,
                     wo3_ref, bo3_ref, out_ref):
    pn = pn_ref[...]
    rterm = (jnp.dot(pn, w1r_ref[...], preferred_element_type=jnp.float32)
             + b1_ref[...])
    en = []
    for k in range(K):
        h = _gelu(jnp.dot(sf_ref[k], w1s_ref[...],
                          preferred_element_type=jnp.float32)
                  + jnp.dot(e_ref[k], w1e_ref[...],
                            preferred_element_type=jnp.float32)
                  + rterm)
        en.append(jnp.dot(h, w2_ref[...], preferred_element_type=jnp.float32)
                  + b2_ref[...])
    agg = (en[0] + en[1]) + en[2]
    hn = _gelu(jnp.dot(pn, wn1p_ref[...], preferred_element_type=jnp.float32)
               + jnp.dot(agg, wn1a_ref[...], preferred_element_type=jnp.float32)
               + bn1_ref[...])
    po = pn + (jnp.dot(hn, wn2_ref[...], preferred_element_type=jnp.float32)
               + bn2_ref[...])
    h1 = _gelu(jnp.dot(po, wo1_ref[...], preferred_element_type=jnp.float32)
               + bo1_ref[...])
    h2 = _gelu(jnp.dot(h1, wo2_ref[...], preferred_element_type=jnp.float32)
               + bo2_ref[...])
    out_ref[...] = (jnp.dot(h2, wo3_ref[...],
                            preferred_element_type=jnp.float32) + bo3_ref[...])


def _layer2_out(pn, sf, e, lw):
    grid = (NQ // QB,)
    wspecs = [pl.BlockSpec(w.shape, lambda i: tuple(0 for _ in w.shape))
              for w in lw]
    out = _pallas_call(
        _layer2_out_body,
        grid=grid,
        in_specs=[
            pl.BlockSpec((QB, ND), lambda i: (i, 0)),
            pl.BlockSpec((K, QB, ND), lambda i: (0, i, 0)),
            pl.BlockSpec((K, QB, ED), lambda i: (0, i, 0)),
        ] + wspecs,
        out_specs=pl.BlockSpec((QB, 3), lambda i: (i, 0)),
        out_shape=jax.ShapeDtypeStruct((NQ, 3), jnp.float32),
    )(pn, sf, e, *lw)
    return out


def kernel(nodes, positions, query_positions, params):
    idx, dsel = _knn(query_positions, positions)

    senders = idx.T.reshape(-1)
    sz = jnp.zeros((BPAD - K * NQ,), jnp.int32)
    idx2d = jnp.concatenate([senders, sz]).reshape(BPAD // CHUNK, CHUNK)
    dsel_t = dsel.T.reshape(K, NQ, 1)

    r2 = lambda b: b.reshape(1, -1)

    g = _gather_rows(nodes, idx2d)[:K * NQ].reshape(K, NQ, ND)
    wbar = jnp.sum(params["enc_W"][0], axis=0, keepdims=True)
    pn, e = _init_enc(g, dsel_t, wbar, r2(params["enc_b"][0]),
                      params["enc_W"][1], r2(params["enc_b"][1]))

    lws = []
    for lp in params["layers"]:
        w1 = lp["eW"][0]
        wn1 = lp["nW"][0]
        lws.append([w1[:ND], w1[ND:2 * ND], w1[2 * ND:], r2(lp["eb"][0]),
                    lp["eW"][1], r2(lp["eb"][1]),
                    wn1[:ND], wn1[ND:], r2(lp["nb"][0]),
                    lp["nW"][1], r2(lp["nb"][1])])

    sf = _gather_rows(pn, idx2d)[:K * NQ].reshape(K, NQ, ND)
    pn, e = _layer(pn, sf, e, lws[0])

    sf = _gather_rows(pn, idx2d)[:K * NQ].reshape(K, NQ, ND)
    ow = [params["out_W"][0], r2(params["out_b"][0]),
          params["out_W"][1], r2(params["out_b"][1]),
          params["out_W"][2], r2(params["out_b"][2])]
    return _layer2_out(pn, sf, e, lws[1] + ow)


# speedup vs baseline: 6.9827x; 1.1498x over previous
import functools

import jax
import jax.numpy as jnp
from jax import lax
from jax.experimental import pallas as pl
from jax.experimental.pallas import tpu as pltpu
from jax.experimental.pallas import tpu_sc as plsc

NQ = 10000
NS = 10000
K = 3
ND = 128
ED = 16
HID = 128
PPAD = 128

_pallas_call = pl.pallas_call

QB_KNN = 200


def _knn_body(qp_ref, spt_ref, idx_ref, dsel_ref):
    qp = qp_ref[...]
    spt = spt_ref[...]
    qb = qp.astype(jnp.bfloat16)
    sbt = spt.astype(jnp.bfloat16)
    sumq = jnp.sum(qp * qp, axis=1, keepdims=True)
    sums = jnp.sum(spt * spt, axis=0, keepdims=True)
    dot = lax.dot_general(qb, sbt, (((1,), (0,)), ((), ())),
                          preferred_element_type=jnp.float32)
    d2 = (sumq + sums) - 2.0 * dot
    iota = lax.broadcasted_iota(jnp.int32, d2.shape, 1)
    dw = jnp.maximum(d2, 1e-12)
    for k in range(K):
        m = jnp.min(dw, axis=1, keepdims=True)
        i = jnp.min(jnp.where(dw == m, iota, jnp.int32(2 ** 30)),
                    axis=1, keepdims=True)
        idx_ref[:, k:k + 1] = i
        dsel_ref[:, k:k + 1] = jnp.sqrt(m)
        if k < K - 1:
            dw = jnp.where(iota == i, jnp.float32(jnp.inf), dw)


def _knn(qpos, spos):
    grid = (NQ // QB_KNN,)
    idx, dsel = _pallas_call(
        _knn_body,
        grid=grid,
        in_specs=[
            pl.BlockSpec((QB_KNN, 3), lambda i: (i, 0)),
            pl.BlockSpec((3, NS), lambda i: (0, 0)),
        ],
        out_specs=[
            pl.BlockSpec((QB_KNN, K), lambda i: (i, 0)),
            pl.BlockSpec((QB_KNN, K), lambda i: (i, 0)),
        ],
        out_shape=[
            jax.ShapeDtypeStruct((NQ, K), jnp.int32),
            jax.ShapeDtypeStruct((NQ, K), jnp.float32),
        ],
    )(qpos, spos.T)
    return idx, dsel


BPAD = 30720
CHUNK = 120


def _gather_rows(table, idx2d):
    info = plsc.get_sparse_core_info()
    nc, nsub = info.num_cores, info.num_subcores
    nw = nc * nsub
    cpw = BPAD // nw // CHUNK
    mesh = plsc.VectorSubcoreMesh(core_axis_name="c", subcore_axis_name="s")

    @functools.partial(
        pl.kernel, mesh=mesh,
        out_type=jax.ShapeDtypeStruct((BPAD, ND), jnp.float32),
        scratch_types=[
            pltpu.VMEM((cpw, CHUNK), jnp.int32),
            pltpu.VMEM((cpw, CHUNK, ND), jnp.float32),
            pltpu.SemaphoreType.DMA,
            pltpu.SemaphoreType.DMA,
        ],
    )
    def gk(table_hbm, idx_hbm, out_hbm, idx_v, rows_v, sem_g, sem_o):
        wid = lax.axis_index("s") * nc + lax.axis_index("c")
        rowbase = wid * cpw
        pltpu.sync_copy(idx_hbm.at[pl.ds(rowbase, cpw)], idx_v)
        gets = [pltpu.async_copy(table_hbm.at[idx_v.at[j]], rows_v.at[j],
                                 sem_g) for j in range(cpw)]
        for c in gets:
            c.wait()
        puts = [pltpu.async_copy(rows_v.at[j],
                                 out_hbm.at[pl.ds((rowbase + j) * CHUNK,
                                                  CHUNK)],
                                 sem_o) for j in range(cpw)]
        for c in puts:
            c.wait()

    return gk(table, idx2d)


QB = 1000


_SQRT_HALF = 0.7071067811865476


def _gelu(x):
    return 0.5 * x * (1.0 + lax.erf(x * _SQRT_HALF))


def _init_enc_body(g_ref, d_ref, wbar_ref, b1_ref, w2_ref, b2_ref,
                   pn_ref, e_ref):
    g = g_ref[...]
    pn_ref[...] = ((g[0] + g[1]) + g[2]) / 3.0
    wbar = wbar_ref[...]
    b1 = b1_ref[...]
    w2 = w2_ref[...]
    b2 = b2_ref[...]
    for k in range(K):
        dk = d_ref[k]
        h = _gelu(dk * wbar + b1)
        e_ref[k] = jnp.dot(h, w2, preferred_element_type=jnp.float32) + b2


def _init_enc(g, dsel_t, wbar, b1, w2, b2):
    grid = (NQ // QB,)
    pn, e = _pallas_call(
        _init_enc_body,
        grid=grid,
        in_specs=[
            pl.BlockSpec((K, QB, ND), lambda i: (0, i, 0)),
            pl.BlockSpec((K, QB, 1), lambda i: (0, i, 0)),
            pl.BlockSpec((1, HID), lambda i: (0, 0)),
            pl.BlockSpec((1, HID), lambda i: (0, 0)),
            pl.BlockSpec((HID, ED), lambda i: (0, 0)),
            pl.BlockSpec((1, ED), lambda i: (0, 0)),
        ],
        out_specs=[
            pl.BlockSpec((QB, ND), lambda i: (i, 0)),
            pl.BlockSpec((K, QB, ED), lambda i: (0, i, 0)),
        ],
        out_shape=[
            jax.ShapeDtypeStruct((NQ, ND), jnp.float32),
            jax.ShapeDtypeStruct((K, NQ, ED), jnp.float32),
        ],
    )(g, dsel_t, wbar, b1, w2, b2)
    return pn, e


def _layer_body(pn_ref, sf_ref, e_ref, w1s_ref, w1r_ref, w1e_ref, b1_ref,
                w2_ref, b2_ref, wn1p_ref, wn1a_ref, bn1_ref, wn2_ref, bn2_ref,
                pno_ref, eo_ref):
    pn = pn_ref[...]
    rterm = (jnp.dot(pn, w1r_ref[...], preferred_element_type=jnp.float32)
             + b1_ref[...])
    en = []
    for k in range(K):
        h = _gelu(jnp.dot(sf_ref[k], w1s_ref[...],
                          preferred_element_type=jnp.float32)
                  + jnp.dot(e_ref[k], w1e_ref[...],
                            preferred_element_type=jnp.float32)
                  + rterm)
        enk = (jnp.dot(h, w2_ref[...], preferred_element_type=jnp.float32)
               + b2_ref[...])
        en.append(enk)
        eo_ref[k] = e_ref[k] + enk
    agg = (en[0] + en[1]) + en[2]
    hn = _gelu(jnp.dot(pn, wn1p_ref[...], preferred_element_type=jnp.float32)
               + jnp.dot(agg, wn1a_ref[...], preferred_element_type=jnp.float32)
               + bn1_ref[...])
    pno_ref[...] = pn + (jnp.dot(hn, wn2_ref[...],
                                 preferred_element_type=jnp.float32)
                         + bn2_ref[...])


def _layer(pn, sf, e, lw):
    grid = (NQ // QB,)
    wspecs = [pl.BlockSpec(w.shape, lambda i: tuple(0 for _ in w.shape))
              for w in lw]
    pno, eo = _pallas_call(
        _layer_body,
        grid=grid,
        in_specs=[
            pl.BlockSpec((QB, ND), lambda i: (i, 0)),
            pl.BlockSpec((K, QB, ND), lambda i: (0, i, 0)),
            pl.BlockSpec((K, QB, ED), lambda i: (0, i, 0)),
        ] + wspecs,
        out_specs=[
            pl.BlockSpec((QB, ND), lambda i: (i, 0)),
            pl.BlockSpec((K, QB, ED), lambda i: (0, i, 0)),
        ],
        out_shape=[
            jax.ShapeDtypeStruct((NQ, ND), jnp.float32),
            jax.ShapeDtypeStruct((K, NQ, ED), jnp.float32),
        ],
    )(pn, sf, e, *lw)
    return pno, eo


def _layer2_out_body(pn_ref, sf_ref, e_ref, w1s_ref, w1r_ref, w1e_ref,
                     b1_ref, w2_ref, b2_ref, wn1p_ref, wn1a_ref, bn1_ref,
                     wn2_ref, bn2_ref, wo1_ref, bo1_ref, wo2_ref, bo2_ref,
                     wo3_ref, bo3_ref, out_ref):
    pn = pn_ref[...]
    rterm = (jnp.dot(pn, w1r_ref[...], preferred_element_type=jnp.float32)
             + b1_ref[...])
    en = []
    for k in range(K):
        h = _gelu(jnp.dot(sf_ref[k], w1s_ref[...],
                          preferred_element_type=jnp.float32)
                  + jnp.dot(e_ref[k], w1e_ref[...],
                            preferred_element_type=jnp.float32)
                  + rterm)
        en.append(jnp.dot(h, w2_ref[...], preferred_element_type=jnp.float32)
                  + b2_ref[...])
    agg = (en[0] + en[1]) + en[2]
    hn = _gelu(jnp.dot(pn, wn1p_ref[...], preferred_element_type=jnp.float32)
               + jnp.dot(agg, wn1a_ref[...], preferred_element_type=jnp.float32)
               + bn1_ref[...])
    po = pn + (jnp.dot(hn, wn2_ref[...], preferred_element_type=jnp.float32)
               + bn2_ref[...])
    h1 = _gelu(jnp.dot(po, wo1_ref[...], preferred_element_type=jnp.float32)
               + bo1_ref[...])
    h2 = _gelu(jnp.dot(h1, wo2_ref[...], preferred_element_type=jnp.float32)
               + bo2_ref[...])
    out_ref[...] = (jnp.dot(h2, wo3_ref[...],
                            preferred_element_type=jnp.float32) + bo3_ref[...])


def _layer2_out(pn, sf, e, lw):
    grid = (NQ // QB,)
    wspecs = [pl.BlockSpec(w.shape, lambda i: tuple(0 for _ in w.shape))
              for w in lw]
    out = _pallas_call(
        _layer2_out_body,
        grid=grid,
        in_specs=[
            pl.BlockSpec((QB, ND), lambda i: (i, 0)),
            pl.BlockSpec((K, QB, ND), lambda i: (0, i, 0)),
            pl.BlockSpec((K, QB, ED), lambda i: (0, i, 0)),
        ] + wspecs,
        out_specs=pl.BlockSpec((QB, 3), lambda i: (i, 0)),
        out_shape=jax.ShapeDtypeStruct((NQ, 3), jnp.float32),
    )(pn, sf, e, *lw)
    return out


def kernel(nodes, positions, query_positions, params):
    idx, dsel = _knn(query_positions, positions)

    senders = idx.T.reshape(-1)
    sz = jnp.zeros((BPAD - K * NQ,), jnp.int32)
    idx2d = jnp.concatenate([senders, sz]).reshape(BPAD // CHUNK, CHUNK)
    dsel_t = dsel.T.reshape(K, NQ, 1)

    r2 = lambda b: b.reshape(1, -1)

    g = _gather_rows(nodes, idx2d)[:K * NQ].reshape(K, NQ, ND)
    wbar = jnp.sum(params["enc_W"][0], axis=0, keepdims=True)
    pn, e = _init_enc(g, dsel_t, wbar, r2(params["enc_b"][0]),
                      params["enc_W"][1], r2(params["enc_b"][1]))

    lws = []
    for lp in params["layers"]:
        w1 = lp["eW"][0]
        wn1 = lp["nW"][0]
        lws.append([w1[:ND], w1[ND:2 * ND], w1[2 * ND:], r2(lp["eb"][0]),
                    lp["eW"][1], r2(lp["eb"][1]),
                    wn1[:ND], wn1[ND:], r2(lp["nb"][0]),
                    lp["nW"][1], r2(lp["nb"][1])])

    sf = _gather_rows(pn, idx2d)[:K * NQ].reshape(K, NQ, ND)
    pn, e = _layer(pn, sf, e, lws[0])

    sf = _gather_rows(pn, idx2d)[:K * NQ].reshape(K, NQ, ND)
    ow = [params["out_W"][0], r2(params["out_b"][0]),
          params["out_W"][1], r2(params["out_b"][1]),
          params["out_W"][2], r2(params["out_b"][2])]
    return _layer2_out(pn, sf, e, lws[1] + ow)


# speedup vs baseline: 7.2029x; 1.0315x over previous
import functools

import jax
import jax.numpy as jnp
from jax import lax
from jax.experimental import pallas as pl
from jax.experimental.pallas import tpu as pltpu
from jax.experimental.pallas import tpu_sc as plsc

NQ = 10000
NS = 10000
K = 3
ND = 128
ED = 16
HID = 128
PPAD = 128

_pallas_call = pl.pallas_call

QB_KNN = 400


def _knn_body(qp_ref, spt_ref, idx_ref, dsel_ref):
    qp = qp_ref[...]
    spt = spt_ref[...]
    qb = qp.astype(jnp.bfloat16)
    sbt = spt.astype(jnp.bfloat16)
    sumq = jnp.sum(qp * qp, axis=1, keepdims=True)
    sums = jnp.sum(spt * spt, axis=0, keepdims=True)
    dot = lax.dot_general(qb, sbt, (((1,), (0,)), ((), ())),
                          preferred_element_type=jnp.float32)
    d2 = (sumq + sums) - 2.0 * dot
    iota = lax.broadcasted_iota(jnp.int32, d2.shape, 1)
    dw = jnp.maximum(d2, 1e-12)
    for k in range(K):
        m = jnp.min(dw, axis=1, keepdims=True)
        i = jnp.min(jnp.where(dw == m, iota, jnp.int32(2 ** 30)),
                    axis=1, keepdims=True)
        idx_ref[:, k:k + 1] = i
        dsel_ref[:, k:k + 1] = jnp.sqrt(m)
        if k < K - 1:
            dw = jnp.where(iota == i, jnp.float32(jnp.inf), dw)


def _knn(qpos, spos):
    grid = (NQ // QB_KNN,)
    idx, dsel = _pallas_call(
        _knn_body,
        grid=grid,
        in_specs=[
            pl.BlockSpec((QB_KNN, 3), lambda i: (i, 0)),
            pl.BlockSpec((3, NS), lambda i: (0, 0)),
        ],
        out_specs=[
            pl.BlockSpec((QB_KNN, K), lambda i: (i, 0)),
            pl.BlockSpec((QB_KNN, K), lambda i: (i, 0)),
        ],
        out_shape=[
            jax.ShapeDtypeStruct((NQ, K), jnp.int32),
            jax.ShapeDtypeStruct((NQ, K), jnp.float32),
        ],
    )(qpos, spos.T)
    return idx, dsel


BPAD = 30720
CHUNK = 120


def _gather_rows(table, idx2d):
    info = plsc.get_sparse_core_info()
    nc, nsub = info.num_cores, info.num_subcores
    nw = nc * nsub
    cpw = BPAD // nw // CHUNK
    mesh = plsc.VectorSubcoreMesh(core_axis_name="c", subcore_axis_name="s")

    @functools.partial(
        pl.kernel, mesh=mesh,
        out_type=jax.ShapeDtypeStruct((BPAD, ND), jnp.float32),
        scratch_types=[
            pltpu.VMEM((cpw, CHUNK), jnp.int32),
            pltpu.VMEM((cpw, CHUNK, ND), jnp.float32),
            pltpu.SemaphoreType.DMA,
            pltpu.SemaphoreType.DMA,
        ],
    )
    def gk(table_hbm, idx_hbm, out_hbm, idx_v, rows_v, sem_g, sem_o):
        wid = lax.axis_index("s") * nc + lax.axis_index("c")
        rowbase = wid * cpw
        pltpu.sync_copy(idx_hbm.at[pl.ds(rowbase, cpw)], idx_v)
        gets = [pltpu.async_copy(table_hbm.at[idx_v.at[j]], rows_v.at[j],
                                 sem_g) for j in range(cpw)]
        for c in gets:
            c.wait()
        puts = [pltpu.async_copy(rows_v.at[j],
                                 out_hbm.at[pl.ds((rowbase + j) * CHUNK,
                                                  CHUNK)],
                                 sem_o) for j in range(cpw)]
        for c in puts:
            c.wait()

    return gk(table, idx2d)


QB = 1000


_SQRT_HALF = 0.7071067811865476


def _gelu(x):
    return 0.5 * x * (1.0 + lax.erf(x * _SQRT_HALF))


def _init_enc_body(g_ref, d_ref, wbar_ref, b1_ref, w2_ref, b2_ref,
                   pn_ref, e_ref):
    g = g_ref[...]
    pn_ref[...] = ((g[0] + g[1]) + g[2]) / 3.0
    wbar = wbar_ref[...]
    b1 = b1_ref[...]
    w2 = w2_ref[...]
    b2 = b2_ref[...]
    for k in range(K):
        dk = d_ref[k]
        h = _gelu(dk * wbar + b1)
        e_ref[k] = jnp.dot(h, w2, preferred_element_type=jnp.float32) + b2


def _init_enc(g, dsel_t, wbar, b1, w2, b2):
    grid = (NQ // QB,)
    pn, e = _pallas_call(
        _init_enc_body,
        grid=grid,
        in_specs=[
            pl.BlockSpec((K, QB, ND), lambda i: (0, i, 0)),
            pl.BlockSpec((K, QB, 1), lambda i: (0, i, 0)),
            pl.BlockSpec((1, HID), lambda i: (0, 0)),
            pl.BlockSpec((1, HID), lambda i: (0, 0)),
            pl.BlockSpec((HID, ED), lambda i: (0, 0)),
            pl.BlockSpec((1, ED), lambda i: (0, 0)),
        ],
        out_specs=[
            pl.BlockSpec((QB, ND), lambda i: (i, 0)),
            pl.BlockSpec((K, QB, ED), lambda i: (0, i, 0)),
        ],
        out_shape=[
            jax.ShapeDtypeStruct((NQ, ND), jnp.float32),
            jax.ShapeDtypeStruct((K, NQ, ED), jnp.float32),
        ],
    )(g, dsel_t, wbar, b1, w2, b2)
    return pn, e


def _layer_body(pn_ref, sf_ref, e_ref, w1s_ref, w1r_ref, w1e_ref, b1_ref,
                w2_ref, b2_ref, wn1p_ref, wn1a_ref, bn1_ref, wn2_ref, bn2_ref,
                pno_ref, eo_ref):
    pn = pn_ref[...]
    rterm = (jnp.dot(pn, w1r_ref[...], preferred_element_type=jnp.float32)
             + b1_ref[...])
    en = []
    for k in range(K):
        h = _gelu(jnp.dot(sf_ref[k], w1s_ref[...],
                          preferred_element_type=jnp.float32)
                  + jnp.dot(e_ref[k], w1e_ref[...],
                            preferred_element_type=jnp.float32)
                  + rterm)
        enk = (jnp.dot(h, w2_ref[...], preferred_element_type=jnp.float32)
               + b2_ref[...])
        en.append(enk)
        eo_ref[k] = e_ref[k] + enk
    agg = (en[0] + en[1]) + en[2]
    hn = _gelu(jnp.dot(pn, wn1p_ref[...], preferred_element_type=jnp.float32)
               + jnp.dot(agg, wn1a_ref[...], preferred_element_type=jnp.float32)
               + bn1_ref[...])
    pno_ref[...] = pn + (jnp.dot(hn, wn2_ref[...],
                                 preferred_element_type=jnp.float32)
                         + bn2_ref[...])


def _layer(pn, sf, e, lw):
    grid = (NQ // QB,)
    wspecs = [pl.BlockSpec(w.shape, lambda i: tuple(0 for _ in w.shape))
              for w in lw]
    pno, eo = _pallas_call(
        _layer_body,
        grid=grid,
        in_specs=[
            pl.BlockSpec((QB, ND), lambda i: (i, 0)),
            pl.BlockSpec((K, QB, ND), lambda i: (0, i, 0)),
            pl.BlockSpec((K, QB, ED), lambda i: (0, i, 0)),
        ] + wspecs,
        out_specs=[
            pl.BlockSpec((QB, ND), lambda i: (i, 0)),
            pl.BlockSpec((K, QB, ED), lambda i: (0, i, 0)),
        ],
        out_shape=[
            jax.ShapeDtypeStruct((NQ, ND), jnp.float32),
            jax.ShapeDtypeStruct((K, NQ, ED), jnp.float32),
        ],
    )(pn, sf, e, *lw)
    return pno, eo


def _layer2_out_body(pn_ref, sf_ref, e_ref, w1s_ref, w1r_ref, w1e_ref,
                     b1_ref, w2_ref, b2_ref, wn1p_ref, wn1a_ref, bn1_ref,
                     wn2_ref, bn2_ref, wo1_ref, bo1_ref, wo2_ref, bo2_ref,
                     wo3_ref, bo3_ref, out_ref):
    pn = pn_ref[...]
    rterm = (jnp.dot(pn, w1r_ref[...], preferred_element_type=jnp.float32)
             + b1_ref[...])
    en = []
    for k in range(K):
        h = _gelu(jnp.dot(sf_ref[k], w1s_ref[...],
                          preferred_element_type=jnp.float32)
                  + jnp.dot(e_ref[k], w1e_ref[...],
                            preferred_element_type=jnp.float32)
                  + rterm)
        en.append(jnp.dot(h, w2_ref[...], preferred_element_type=jnp.float32)
                  + b2_ref[...])
    agg = (en[0] + en[1]) + en[2]
    hn = _gelu(jnp.dot(pn, wn1p_ref[...], preferred_element_type=jnp.float32)
               + jnp.dot(agg, wn1a_ref[...], preferred_element_type=jnp.float32)
               + bn1_ref[...])
    po = pn + (jnp.dot(hn, wn2_ref[...], preferred_element_type=jnp.float32)
               + bn2_ref[...])
    h1 = _gelu(jnp.dot(po, wo1_ref[...], preferred_element_type=jnp.float32)
               + bo1_ref[...])
    h2 = _gelu(jnp.dot(h1, wo2_ref[...], preferred_element_type=jnp.float32)
               + bo2_ref[...])
    out_ref[...] = (jnp.dot(h2, wo3_ref[...],
                            preferred_element_type=jnp.float32) + bo3_ref[...])


def _layer2_out(pn, sf, e, lw):
    grid = (NQ // QB,)
    wspecs = [pl.BlockSpec(w.shape, lambda i: tuple(0 for _ in w.shape))
              for w in lw]
    out = _pallas_call(
        _layer2_out_body,
        grid=grid,
        in_specs=[
            pl.BlockSpec((QB, ND), lambda i: (i, 0)),
            pl.BlockSpec((K, QB, ND), lambda i: (0, i, 0)),
            pl.BlockSpec((K, QB, ED), lambda i: (0, i, 0)),
        ] + wspecs,
        out_specs=pl.BlockSpec((QB, 3), lambda i: (i, 0)),
        out_shape=jax.ShapeDtypeStruct((NQ, 3), jnp.float32),
    )(pn, sf, e, *lw)
    return out


def kernel(nodes, positions, query_positions, params):
    idx, dsel = _knn(query_positions, positions)

    senders = idx.T.reshape(-1)
    sz = jnp.zeros((BPAD - K * NQ,), jnp.int32)
    idx2d = jnp.concatenate([senders, sz]).reshape(BPAD // CHUNK, CHUNK)
    dsel_t = dsel.T.reshape(K, NQ, 1)

    r2 = lambda b: b.reshape(1, -1)

    g = _gather_rows(nodes, idx2d)[:K * NQ].reshape(K, NQ, ND)
    wbar = jnp.sum(params["enc_W"][0], axis=0, keepdims=True)
    pn, e = _init_enc(g, dsel_t, wbar, r2(params["enc_b"][0]),
                      params["enc_W"][1], r2(params["enc_b"][1]))

    lws = []
    for lp in params["layers"]:
        w1 = lp["eW"][0]
        wn1 = lp["nW"][0]
        lws.append([w1[:ND], w1[ND:2 * ND], w1[2 * ND:], r2(lp["eb"][0]),
                    lp["eW"][1], r2(lp["eb"][1]),
                    wn1[:ND], wn1[ND:], r2(lp["nb"][0]),
                    lp["nW"][1], r2(lp["nb"][1])])

    sf = _gather_rows(pn, idx2d)[:K * NQ].reshape(K, NQ, ND)
    pn, e = _layer(pn, sf, e, lws[0])

    sf = _gather_rows(pn, idx2d)[:K * NQ].reshape(K, NQ, ND)
    ow = [params["out_W"][0], r2(params["out_b"][0]),
          params["out_W"][1], r2(params["out_b"][1]),
          params["out_W"][2], r2(params["out_b"][2])]
    return _layer2_out(pn, sf, e, lws[1] + ow)
